# Initial kernel scaffold; baseline (speedup 1.0000x reference)
import jax
import jax.numpy as jnp
from jax.experimental import pallas as pl


def kernel(x, W):
    raise NotImplementedError("write your pallas kernel here")


# speedup vs baseline: 1.0934x; 1.0934x over previous
import functools

import jax
import jax.numpy as jnp
from jax import lax
from jax.experimental import pallas as pl
from jax.experimental.pallas import tpu as pltpu
from jax.experimental.pallas import tpu_sc as plsc

NC = 2
NS = 16
NW = NC * NS
IDX_W = 128
G = 8


@functools.partial(jax.jit, static_argnames=("nrows", "emb"))
def _emb_lookup(xr, W, *, nrows, emb):
    rows_per_w = nrows // NW
    n_chunks = rows_per_w // G
    mesh = plsc.VectorSubcoreMesh(
        core_axis_name="c", subcore_axis_name="s", num_cores=NC, num_subcores=NS
    )

    @functools.partial(
        pl.kernel,
        out_type=jax.ShapeDtypeStruct((nrows, IDX_W, emb), jnp.float32),
        mesh=mesh,
        scratch_types=[
            pltpu.VMEM((G, IDX_W), jnp.int32),
            pltpu.VMEM((G, IDX_W, emb), jnp.float32),
            pltpu.SemaphoreType.DMA,
        ],
        compiler_params=pltpu.CompilerParams(use_tc_tiling_on_sc=False),
    )
    def k(x_hbm, w_hbm, out_hbm, idx_v, rows_v, sem):
        wid = lax.axis_index("s") * NC + lax.axis_index("c")
        row0 = wid * rows_per_w

        def chunk(c, carry):
            base = row0 + c * G
            pltpu.sync_copy(x_hbm.at[pl.ds(base, G)], idx_v)
            cps = [
                pltpu.async_copy(w_hbm.at[idx_v.at[j]], rows_v.at[j], sem)
                for j in range(G)
            ]
            for cp in cps:
                cp.wait()
            pltpu.sync_copy(rows_v, out_hbm.at[pl.ds(base, G)])
            return carry

        lax.fori_loop(0, n_chunks, chunk, 0)

    return k(xr, W)


def kernel(x, W):
    if x.ndim > 1:
        shape = (x.shape[0], x.shape[1], -1)
    else:
        shape = (x.shape[0], 1, -1)
    n = x.size
    emb = W.shape[1]
    xr = x.reshape(n // IDX_W, IDX_W).astype(jnp.int32)
    out = _emb_lookup(xr, W, nrows=n // IDX_W, emb=emb)
    return out.reshape(*shape)


# speedup vs baseline: 1.1276x; 1.0312x over previous
import functools

import jax
import jax.numpy as jnp
from jax import lax
from jax.experimental import pallas as pl
from jax.experimental.pallas import tpu as pltpu
from jax.experimental.pallas import tpu_sc as plsc

NC = 2
NS = 16
NW = NC * NS
CH = 1280


@functools.partial(jax.jit, static_argnames=("n", "emb"))
def _emb_lookup(xf, W, *, n, emb):
    n_per_w = n // NW
    n_chunks = n_per_w // CH
    n_pairs = n_chunks // 2
    mesh = plsc.VectorSubcoreMesh(
        core_axis_name="c", subcore_axis_name="s", num_cores=NC, num_subcores=NS
    )

    @functools.partial(
        pl.kernel,
        out_type=jax.ShapeDtypeStruct((n, emb), jnp.float32),
        mesh=mesh,
        scratch_types=[
            pltpu.VMEM((n_per_w,), jnp.int32),
            pltpu.VMEM((CH, emb), jnp.float32),
            pltpu.VMEM((CH, emb), jnp.float32),
            pltpu.SemaphoreType.DMA,
            pltpu.SemaphoreType.DMA,
            pltpu.SemaphoreType.DMA,
            pltpu.SemaphoreType.DMA,
        ],
        compiler_params=pltpu.CompilerParams(use_tc_tiling_on_sc=False),
    )
    def k(x_hbm, w_hbm, out_hbm, idx_all, rows0, rows1, gsem0, gsem1, osem0, osem1):
        wid = lax.axis_index("s") * NC + lax.axis_index("c")
        base = wid * n_per_w
        pltpu.sync_copy(x_hbm.at[pl.ds(base, n_per_w)], idx_all)

        def fire_gather(c, rows, gsem):
            return pltpu.async_copy(w_hbm.at[idx_all.at[pl.ds(c * CH, CH)]], rows, gsem)

        def wait_gather(rows, gsem):
            pltpu.make_async_copy(
                w_hbm.at[idx_all.at[pl.ds(0, CH)]], rows, gsem
            ).wait()

        def fire_out(c, rows, osem):
            return pltpu.async_copy(rows, out_hbm.at[pl.ds(base + c * CH, CH)], osem)

        def wait_out(c, rows, osem):
            pltpu.make_async_copy(
                rows, out_hbm.at[pl.ds(base + c * CH, CH)], osem
            ).wait()

        fire_gather(0, rows0, gsem0)
        fire_gather(1, rows1, gsem1)
        wait_gather(rows0, gsem0)
        fire_out(0, rows0, osem0)

        def pair(k_, carry):
            c0 = 2 * k_
            wait_out(c0 - 2, rows0, osem0)
            fire_gather(c0, rows0, gsem0)
            wait_gather(rows1, gsem1)
            fire_out(c0 - 1, rows1, osem1)

            wait_out(c0 - 1, rows1, osem1)
            fire_gather(c0 + 1, rows1, gsem1)
            wait_gather(rows0, gsem0)
            fire_out(c0, rows0, osem0)
            return carry

        lax.fori_loop(1, n_pairs, pair, 0)

        wait_gather(rows1, gsem1)
        wait_out(n_chunks - 2, rows0, osem0)
        fire_out(n_chunks - 1, rows1, osem1)
        wait_out(n_chunks - 1, rows1, osem1)

    return k(xf, W)


def kernel(x, W):
    if x.ndim > 1:
        shape = (x.shape[0], x.shape[1], -1)
    else:
        shape = (x.shape[0], 1, -1)
    n = x.size
    emb = W.shape[1]
    xf = x.reshape(n).astype(jnp.int32)
    out = _emb_lookup(xf, W, n=n, emb=emb)
    return out.reshape(*shape)
